# baseline (device time: 186064 ns/iter reference)
import jax
import jax.numpy as jnp
from jax import lax
from jax.experimental import pallas as pl
from jax.experimental.pallas import tpu as pltpu

MESH = pl.DeviceIdType.MESH

CH = 128
N_ZCH = 22
N_PCH = 42
ZROWS = 2816


def kernel(x):
    m, n = x.shape
    Q = m // 4

    def body(x_hbm, out_hbm, zrecv, xl, red,
             zsend_sems, zrecv_sems, psend_sems, precv_sems, lcopy_sems):
        mx = lax.axis_index("x")
        my = lax.axis_index("y")
        mz = lax.axis_index("z")
        q = 2 * mx + my
        p = 2 * (1 - mx) + (1 - my)
        qx = 2 * (1 - mx) + my
        qy = 2 * mx + (1 - my)

        x_nbr = (1 - mx, my, mz)
        y_nbr = (mx, 1 - my, mz)
        z_par = (mx, my, 1 - mz)

        zchunks = [(i * CH, q * Q + i * CH) for i in range(16)] + [
            (2048 + j * CH, p * Q + 1280 + j * CH) for j in range(6)
        ]

        zs = []
        for i, (o, g) in enumerate(zchunks):
            r = pltpu.make_async_remote_copy(
                src_ref=x_hbm.at[pl.ds(g, CH)],
                dst_ref=zrecv.at[pl.ds(o, CH)],
                send_sem=zsend_sems.at[i],
                recv_sem=zrecv_sems.at[i],
                device_id=z_par,
                device_id_type=MESH,
            )
            r.start()
            zs.append(r)

        cp_q = pltpu.make_async_copy(
            x_hbm.at[pl.ds(q * Q, Q)], xl.at[pl.ds(0, Q)], lcopy_sems.at[22]
        )
        cp_q.start()
        cp_p = pltpu.make_async_copy(
            x_hbm.at[pl.ds(p * Q + 1280, 768)],
            xl.at[pl.ds(2048, 768)],
            lcopy_sems.at[23],
        )
        cp_p.start()
        cp_q.wait()
        cp_p.wait()

        sends = []
        local_cps = []

        def fwd(k, src_ref, g_dst, tgt):
            r = pltpu.make_async_remote_copy(
                src_ref=src_ref,
                dst_ref=out_hbm.at[pl.ds(g_dst, CH)],
                send_sem=psend_sems.at[k],
                recv_sem=precv_sems.at[k],
                device_id=tgt,
                device_id_type=MESH,
            )
            r.start()
            sends.append(r)

        def recv_wait(k, g_dst, src_dev):
            r = pltpu.make_async_remote_copy(
                src_ref=red.at[pl.ds(0, CH)],
                dst_ref=out_hbm.at[pl.ds(g_dst, CH)],
                send_sem=psend_sems.at[k],
                recv_sem=precv_sems.at[k],
                device_id=src_dev,
                device_id_type=MESH,
            )
            r.wait_recv()

        def consume_z(i):
            o, g = zchunks[i]
            zs[i].wait_recv()
            red[o:o + CH, :] = xl[o:o + CH, :] + zrecv[o:o + CH, :]
            cp = pltpu.make_async_copy(
                red.at[pl.ds(o, CH)], out_hbm.at[pl.ds(g, CH)],
                lcopy_sems.at[i],
            )
            cp.start()
            local_cps.append(cp)

        for i in range(5):
            consume_z(i)
            fwd(i, red.at[pl.ds(i * CH, CH)], q * Q + i * CH, x_nbr)
        for i in range(5):
            recv_wait(i, qx * Q + i * CH, x_nbr)
            fwd(22 + i, out_hbm.at[pl.ds(qx * Q + i * CH, CH)],
                qx * Q + i * CH, y_nbr)
            consume_z(5 + i)
            fwd(5 + i, red.at[pl.ds(640 + i * CH, CH)],
                q * Q + 640 + i * CH, y_nbr)
        for i in range(5):
            recv_wait(22 + i, p * Q + i * CH, y_nbr)
            fwd(32 + i, out_hbm.at[pl.ds(p * Q + i * CH, CH)],
                p * Q + i * CH, x_nbr)
        for i in range(5):
            recv_wait(5 + i, qy * Q + 640 + i * CH, y_nbr)
            fwd(27 + i, out_hbm.at[pl.ds(qy * Q + 640 + i * CH, CH)],
                qy * Q + 640 + i * CH, x_nbr)
        for i in range(3):
            consume_z(10 + i)
            fwd(10 + i, red.at[pl.ds(1280 + i * CH, CH)],
                q * Q + 1280 + i * CH, x_nbr)
        for i in range(3):
            consume_z(13 + i)
            fwd(13 + i, red.at[pl.ds(1664 + i * CH, CH)],
                q * Q + 1664 + i * CH, y_nbr)
        for i in range(5):
            recv_wait(27 + i, p * Q + 640 + i * CH, x_nbr)
            fwd(37 + i, out_hbm.at[pl.ds(p * Q + 640 + i * CH, CH)],
                p * Q + 640 + i * CH, y_nbr)
        for i in range(3):
            consume_z(16 + i)
            fwd(16 + i, red.at[pl.ds(2048 + i * CH, CH)],
                p * Q + 1280 + i * CH, x_nbr)
        for i in range(3):
            consume_z(19 + i)
            fwd(19 + i, red.at[pl.ds(2432 + i * CH, CH)],
                p * Q + 1664 + i * CH, y_nbr)

        for i in range(3):
            recv_wait(10 + i, qx * Q + 1280 + i * CH, x_nbr)
        for i in range(3):
            recv_wait(13 + i, qy * Q + 1664 + i * CH, y_nbr)
        for i in range(3):
            recv_wait(16 + i, qy * Q + 1280 + i * CH, x_nbr)
        for i in range(3):
            recv_wait(19 + i, qx * Q + 1664 + i * CH, y_nbr)
        for i in range(5):
            recv_wait(32 + i, qy * Q + i * CH, x_nbr)
        for i in range(5):
            recv_wait(37 + i, qx * Q + 640 + i * CH, y_nbr)

        for r in zs:
            r.wait_send()
        for r in sends:
            r.wait_send()
        for cp in local_cps:
            cp.wait()

    out = pl.pallas_call(
        body,
        out_shape=jax.ShapeDtypeStruct((m, n), x.dtype),
        in_specs=[pl.BlockSpec(memory_space=pl.ANY)],
        out_specs=pl.BlockSpec(memory_space=pl.ANY),
        scratch_shapes=[
            pltpu.VMEM((ZROWS, n), x.dtype),
            pltpu.VMEM((ZROWS, n), x.dtype),
            pltpu.VMEM((ZROWS, n), x.dtype),
            pltpu.SemaphoreType.DMA((N_ZCH,)),
            pltpu.SemaphoreType.DMA((N_ZCH,)),
            pltpu.SemaphoreType.DMA((N_PCH,)),
            pltpu.SemaphoreType.DMA((N_PCH,)),
            pltpu.SemaphoreType.DMA((24,)),
        ],
        compiler_params=pltpu.CompilerParams(
            vmem_limit_bytes=64 * 1024 * 1024,
        ),
    )(x)
    return out


# device time: 182344 ns/iter; 1.0204x vs baseline; 1.0204x over previous
import jax
import jax.numpy as jnp
from jax import lax
from jax.experimental import pallas as pl
from jax.experimental.pallas import tpu as pltpu

MESH = pl.DeviceIdType.MESH

_SCHED = [
    ("A", [128, 128, 256, 256, 256]),
    ("B1", [256, 256]),
    ("B2", [256, 256]),
    ("PB1", [256, 256]),
    ("PB2", [256, 128, 128]),
]
_REGION_BUF0 = {"A": 0, "B1": 1024, "B2": 1536, "PB1": 2048, "PB2": 2560}

N_ZCH = sum(len(s) for _, s in _SCHED)
N_PCH = sum(len(s) * (3 if k == "A" else 1) for k, s in _SCHED)


def kernel(x):
    m, n = x.shape
    Q = m // 4

    def body(x_hbm, out_hbm, zrecv, xl, red,
             zsend_sems, zrecv_sems, psend_sems, precv_sems, lcopy_sems):
        mx = lax.axis_index("x")
        my = lax.axis_index("y")
        mz = lax.axis_index("z")
        q = 2 * mx + my
        p = 2 * (1 - mx) + (1 - my)
        qx = 2 * (1 - mx) + my
        qy = 2 * mx + (1 - my)

        x_nbr = (1 - mx, my, mz)
        y_nbr = (mx, 1 - my, mz)
        diag = (1 - mx, 1 - my, mz)
        z_par = (mx, my, 1 - mz)

        def gq(quarter, o):
            return quarter * Q + o

        def gp(quarter_p, o):
            return quarter_p * Q + (o - 2048) + 1024

        chunks = []
        for kind, sizes in _SCHED:
            o = _REGION_BUF0[kind]
            for sz in sizes:
                chunks.append((kind, o, sz))
                o += sz

        def my_g(o):
            return gq(q, o) if o < 2048 else gp(p, o)

        zs = []
        for i, (kind, o, sz) in enumerate(chunks):
            g = my_g(o)
            r = pltpu.make_async_remote_copy(
                src_ref=x_hbm.at[pl.ds(g, sz)],
                dst_ref=zrecv.at[pl.ds(o, sz)],
                send_sem=zsend_sems.at[i],
                recv_sem=zrecv_sems.at[i],
                device_id=z_par,
                device_id_type=MESH,
            )
            r.start()
            zs.append(r)

        cp_q = pltpu.make_async_copy(
            x_hbm.at[pl.ds(q * Q, Q)], xl.at[pl.ds(0, Q)],
            lcopy_sems.at[N_ZCH],
        )
        cp_q.start()
        cp_p = pltpu.make_async_copy(
            x_hbm.at[pl.ds(p * Q + 1024, 1024)],
            xl.at[pl.ds(2048, 1024)],
            lcopy_sems.at[N_ZCH + 1],
        )
        cp_p.start()
        cp_q.wait()
        cp_p.wait()

        kind_targets = {
            "A": [x_nbr, y_nbr, diag],
            "B1": [x_nbr],
            "B2": [y_nbr],
            "PB1": [x_nbr],
            "PB2": [y_nbr],
        }

        sends = []
        local_cps = []
        slot = 0
        for i, (kind, o, sz) in enumerate(chunks):
            g = my_g(o)
            zs[i].wait_recv()
            red[o:o + sz, :] = xl[o:o + sz, :] + zrecv[o:o + sz, :]
            cp = pltpu.make_async_copy(
                red.at[pl.ds(o, sz)], out_hbm.at[pl.ds(g, sz)],
                lcopy_sems.at[i],
            )
            cp.start()
            local_cps.append(cp)
            for tgt in kind_targets[kind]:
                r = pltpu.make_async_remote_copy(
                    src_ref=red.at[pl.ds(o, sz)],
                    dst_ref=out_hbm.at[pl.ds(g, sz)],
                    send_sem=psend_sems.at[slot],
                    recv_sem=precv_sems.at[slot],
                    device_id=tgt,
                    device_id_type=MESH,
                )
                r.start()
                sends.append(r)
                slot += 1

        kind_sources = {
            "A": [(x_nbr, "q", qx), (y_nbr, "q", qy), (diag, "q", p)],
            "B1": [(x_nbr, "q", qx)],
            "B2": [(y_nbr, "q", qy)],
            "PB1": [(x_nbr, "p", qy)],
            "PB2": [(y_nbr, "p", qx)],
        }
        slot = 0
        for kind, o, sz in chunks:
            for src_dev, reg, quarter in kind_sources[kind]:
                g = gq(quarter, o) if reg == "q" else gp(quarter, o)
                r = pltpu.make_async_remote_copy(
                    src_ref=red.at[pl.ds(0, sz)],
                    dst_ref=out_hbm.at[pl.ds(g, sz)],
                    send_sem=psend_sems.at[slot],
                    recv_sem=precv_sems.at[slot],
                    device_id=src_dev,
                    device_id_type=MESH,
                )
                r.wait_recv()
                slot += 1

        for r in zs:
            r.wait_send()
        for r in sends:
            r.wait_send()
        for cp in local_cps:
            cp.wait()

    out = pl.pallas_call(
        body,
        out_shape=jax.ShapeDtypeStruct((m, n), x.dtype),
        in_specs=[pl.BlockSpec(memory_space=pl.ANY)],
        out_specs=pl.BlockSpec(memory_space=pl.ANY),
        scratch_shapes=[
            pltpu.VMEM((3072, n), x.dtype),
            pltpu.VMEM((3072, n), x.dtype),
            pltpu.VMEM((3072, n), x.dtype),
            pltpu.SemaphoreType.DMA((N_ZCH,)),
            pltpu.SemaphoreType.DMA((N_ZCH,)),
            pltpu.SemaphoreType.DMA((N_PCH,)),
            pltpu.SemaphoreType.DMA((N_PCH,)),
            pltpu.SemaphoreType.DMA((N_ZCH + 2,)),
        ],
        compiler_params=pltpu.CompilerParams(
            vmem_limit_bytes=64 * 1024 * 1024,
        ),
    )(x)
    return out
